# baseline (speedup 1.0000x reference)
import functools

import jax
import jax.numpy as jnp
from jax import lax
from jax.experimental import pallas as pl
from jax.experimental.pallas import tpu as pltpu
from jax.experimental.pallas import tpu_sc as plsc

NUM_CORES = 1
NUM_SUBCORES = 16
NUM_WORKERS = NUM_CORES * NUM_SUBCORES


def kernel(entity_ids, entity_table):
    (batch,) = entity_ids.shape
    vocab, dim = entity_table.shape
    per_worker = batch // NUM_WORKERS

    ids32 = entity_ids.astype(jnp.int32)
    table_cols = [entity_table[:, d] for d in range(dim)]

    mesh = plsc.VectorSubcoreMesh(
        core_axis_name="c",
        subcore_axis_name="s",
        num_cores=NUM_CORES,
        num_subcores=NUM_SUBCORES,
    )

    @functools.partial(
        pl.kernel,
        out_type=tuple(
            jax.ShapeDtypeStruct((batch,), jnp.float32) for _ in range(dim)
        ),
        mesh=mesh,
        compiler_params=pltpu.CompilerParams(needs_layout_passes=False),
        scratch_types=[pltpu.VMEM((per_worker,), jnp.int32)]
        + [pltpu.VMEM((per_worker,), jnp.float32) for _ in range(dim)]
        + [pltpu.SemaphoreType.DMA],
    )
    def gather_kernel(ids_hbm, t0_hbm, t1_hbm, t2_hbm,
                      c0_hbm, c1_hbm, c2_hbm,
                      ids_v, c0_v, c1_v, c2_v, sem):
        tcols_hbm = (t0_hbm, t1_hbm, t2_hbm)
        cols_hbm = (c0_hbm, c1_hbm, c2_hbm)
        cols_v = (c0_v, c1_v, c2_v)
        wid = lax.axis_index("s") * NUM_CORES + lax.axis_index("c")
        base = wid * per_worker
        pltpu.async_copy(ids_hbm.at[pl.ds(base, per_worker)], ids_v, sem).wait()
        gathers = [
            pltpu.async_copy(tcols_hbm[d].at[ids_v], cols_v[d], sem)
            for d in range(dim)
        ]
        for g in gathers:
            g.wait()
        out_copies = [
            pltpu.async_copy(cols_v[d], cols_hbm[d].at[pl.ds(base, per_worker)], sem)
            for d in range(dim)
        ]
        for c in out_copies:
            c.wait()

    cols = gather_kernel(ids32, *table_cols)
    return jnp.stack(cols, axis=1)


# speedup vs baseline: 10.0817x; 10.0817x over previous
import functools

import jax
import jax.numpy as jnp
from jax import lax
from jax.experimental import pallas as pl
from jax.experimental.pallas import tpu as pltpu
from jax.experimental.pallas import tpu_sc as plsc

NUM_CORES = 1
NUM_SUBCORES = 16
LANES = 16
NUM_WORKERS = NUM_CORES * NUM_SUBCORES


def kernel(entity_ids, entity_table):
    (batch,) = entity_ids.shape
    vocab, dim = entity_table.shape
    per_worker = batch // NUM_WORKERS

    ids32 = entity_ids.astype(jnp.int32)
    table_cols = [entity_table[:, d] for d in range(dim)]

    mesh = plsc.VectorSubcoreMesh(
        core_axis_name="c",
        subcore_axis_name="s",
        num_cores=NUM_CORES,
        num_subcores=NUM_SUBCORES,
    )

    @functools.partial(
        pl.kernel,
        out_type=tuple(
            jax.ShapeDtypeStruct((batch,), jnp.float32) for _ in range(dim)
        ),
        mesh=mesh,
        compiler_params=pltpu.CompilerParams(needs_layout_passes=False),
        scratch_types=[pltpu.VMEM((per_worker,), jnp.int32)]
        + [pltpu.VMEM((vocab,), jnp.float32) for _ in range(dim)]
        + [pltpu.VMEM((per_worker,), jnp.float32) for _ in range(dim)]
        + [pltpu.SemaphoreType.DMA],
    )
    def gather_kernel(ids_hbm, t0_hbm, t1_hbm, t2_hbm,
                      c0_hbm, c1_hbm, c2_hbm,
                      ids_v, t0_v, t1_v, t2_v, c0_v, c1_v, c2_v, sem):
        tcols_hbm = (t0_hbm, t1_hbm, t2_hbm)
        tcols_v = (t0_v, t1_v, t2_v)
        cols_hbm = (c0_hbm, c1_hbm, c2_hbm)
        cols_v = (c0_v, c1_v, c2_v)
        wid = lax.axis_index("s") * NUM_CORES + lax.axis_index("c")
        base = wid * per_worker
        in_copies = [
            pltpu.async_copy(ids_hbm.at[pl.ds(base, per_worker)], ids_v, sem)
        ] + [
            pltpu.async_copy(tcols_hbm[d], tcols_v[d], sem) for d in range(dim)
        ]
        for c in in_copies:
            c.wait()

        @plsc.parallel_loop(0, per_worker, LANES, unroll=8)
        def body(i):
            rows = ids_v[pl.ds(i, LANES)]
            for d in range(dim):
                cols_v[d][pl.ds(i, LANES)] = plsc.load_gather(
                    tcols_v[d], [rows]
                )

        out_copies = [
            pltpu.async_copy(cols_v[d], cols_hbm[d].at[pl.ds(base, per_worker)], sem)
            for d in range(dim)
        ]
        for c in out_copies:
            c.wait()

    cols = gather_kernel(ids32, *table_cols)
    return jnp.stack(cols, axis=1)
